# baseline (speedup 1.0000x reference)
import functools

import jax
import jax.numpy as jnp
from jax import lax
from jax.experimental import pallas as pl
from jax.experimental.pallas import tpu as pltpu
from jax.experimental.pallas import tpu_sc as plsc

N_NODES = 10000
N_EDGES = 160000
E_TOT = N_EDGES + N_NODES
HEADS = 4
NEG_SLOPE = 0.2
NPAD = 10240
NW = 32
DPW = NPAD // NW
START_LEN = NW * DPW + 32
SRC_LEN = ((E_TOT + 31) // 32) * 32 + 32


def _mm_pair(h, wl, bl, wr, br):
    m, k = h.shape
    n = wl.shape[1]
    rb = 1024
    cb = min(2048, n)

    def body(x_ref, wl_ref, bl_ref, wr_ref, br_ref, ol_ref, or_ref):
        x = x_ref[...]
        ol_ref[...] = (
            jnp.dot(x, wl_ref[...], preferred_element_type=jnp.float32)
            + bl_ref[...]
        )
        or_ref[...] = (
            jnp.dot(x, wr_ref[...], preferred_element_type=jnp.float32)
            + br_ref[...]
        )

    out = pl.pallas_call(
        body,
        grid=(m // rb, n // cb),
        in_specs=[
            pl.BlockSpec((rb, k), lambda i, j: (i, 0)),
            pl.BlockSpec((k, cb), lambda i, j: (0, j)),
            pl.BlockSpec((1, cb), lambda i, j: (0, j)),
            pl.BlockSpec((k, cb), lambda i, j: (0, j)),
            pl.BlockSpec((1, cb), lambda i, j: (0, j)),
        ],
        out_specs=[
            pl.BlockSpec((rb, cb), lambda i, j: (i, j)),
            pl.BlockSpec((rb, cb), lambda i, j: (i, j)),
        ],
        out_shape=[jax.ShapeDtypeStruct((m, n), jnp.float32)] * 2,
    )(h, wl, bl, wr, br)
    return out


def _log_softmax2(h):
    def body(x_ref, o_ref):
        x = x_ref[...]
        x0 = x[:, 0:1]
        x1 = x[:, 1:2]
        mx = jnp.maximum(x0, x1)
        ls = mx + jnp.log(jnp.exp(x0 - mx) + jnp.exp(x1 - mx))
        o_ref[...] = x - ls

    return pl.pallas_call(
        body,
        grid=(NPAD // 1024,),
        in_specs=[pl.BlockSpec((1024, 32), lambda i: (i, 0))],
        out_specs=pl.BlockSpec((1024, 32), lambda i: (i, 0)),
        out_shape=jax.ShapeDtypeStruct((NPAD, 32), jnp.float32),
    )(h)


@functools.lru_cache(maxsize=None)
def _make_edge_kernel(cp: int, do_relu: bool):
    hcp = HEADS * cp
    nbc = cp // 16
    dbuf = hcp <= 2048
    info = plsc.get_sparse_core_info()
    nc = info.num_cores
    mesh = plsc.VectorSubcoreMesh(core_axis_name="c", subcore_axis_name="s")
    a_pos = 0.5 * (1.0 + NEG_SLOPE)
    a_neg = 0.5 * (1.0 - NEG_SLOPE)
    xlg1_shape = (16, hcp) if dbuf else (16, 128)

    @functools.partial(
        pl.kernel,
        out_type=jax.ShapeDtypeStruct((NPAD + NW, cp), jnp.float32),
        mesh=mesh,
        compiler_params=pltpu.CompilerParams(needs_layout_passes=False),
        scratch_types=[
            pltpu.VMEM((DPW + 32,), jnp.int32),
            pltpu.VMEM((HEADS, cp), jnp.float32),
            pltpu.VMEM((cp,), jnp.float32),
            pltpu.VMEM((32,), jnp.int32),
            pltpu.VMEM((32,), jnp.int32),
            pltpu.VMEM((16, hcp), jnp.float32),
            pltpu.VMEM(xlg1_shape, jnp.float32),
            pltpu.VMEM((hcp,), jnp.float32),
            pltpu.VMEM((hcp,), jnp.float32),
            pltpu.VMEM((hcp,), jnp.float32),
            pltpu.VMEM((cp,), jnp.float32),
            pltpu.VMEM((cp,), jnp.float32),
            pltpu.VMEM((256,), jnp.float32),
            pltpu.SemaphoreType.DMA,
            pltpu.SemaphoreType.DMA,
            pltpu.SemaphoreType.DMA,
            pltpu.SemaphoreType.DMA,
            pltpu.SemaphoreType.DMA,
            pltpu.SemaphoreType.DMA,
            pltpu.SemaphoreType.DMA,
            pltpu.SemaphoreType.DMA,
        ],
    )
    def ek(xl_hbm, xr_hbm, src_hbm, start_hbm, att_hbm, b_hbm, out_hbm,
           startv, attv, biasv, idxb0, idxb1, xlg0, xlg1, xrv0, xrv1,
           accv, outr0, outr1, partsv,
           isem0, isem1, rsem0, rsem1, xsem0, xsem1, osem0, osem1):
        wid = lax.axis_index("s") * nc + lax.axis_index("c")
        d0 = wid * DPW
        trash = NPAD + wid
        pltpu.sync_copy(start_hbm.at[pl.ds(d0, DPW + 32)], startv)
        pltpu.sync_copy(att_hbm, attv)
        pltpu.sync_copy(b_hbm, biasv)
        lane = lax.iota(jnp.int32, 16)
        zidx = jnp.zeros((16,), jnp.int32)
        dnums = lax.GatherDimensionNumbers(
            offset_dims=(), collapsed_slice_dims=(0,), start_index_map=(0,))

        idxbs = (idxb0, idxb1)
        xlgs = (xlg0, xlg1) if dbuf else (xlg0, xlg0)
        xrvs = (xrv0, xrv1)
        outrs = (outr0, outr1)
        isems = (isem0, isem1)
        rsems = (rsem0, rsem1)
        xsems = (xsem0, xsem1)
        osems = (osem0, osem1)

        def _shuf(v, idx):
            return lax.gather(
                v, idx.reshape(16, 1), dnums, slice_sizes=(1,),
                mode=lax.GatherScatterMode.PROMISE_IN_BOUNDS)

        def _bsum(v):
            for kk in (8, 4, 2, 1):
                v = v + _shuf(v, lane ^ kk)
            return v

        def _bmax(v):
            for kk in (8, 4, 2, 1):
                v = jnp.maximum(v, _shuf(v, lane ^ kk))
            return v

        def issue_idx(e0, s):
            eba = pl.multiple_of((e0 // 8) * 8, 8)
            pltpu.async_copy(src_hbm.at[pl.ds(eba, 32)], idxbs[s], isems[s])

        def wait_idx(s):
            pltpu.make_async_copy(
                src_hbm.at[pl.ds(0, 32)], idxbs[s], isems[s]).wait()

        def issue_rows(idx, s):
            pltpu.async_copy(xl_hbm.at[idx], xlgs[s], rsems[s])

        def wait_rows(s):
            pltpu.make_async_copy(xl_hbm.at[zidx], xlgs[s], rsems[s]).wait()

        def issue_xr(d, s):
            pltpu.async_copy(xr_hbm.at[d], xrvs[s], xsems[s])

        def wait_xr(s):
            pltpu.make_async_copy(xr_hbm.at[0], xrvs[s], xsems[s]).wait()

        def issue_out(addr, s):
            pltpu.async_copy(outrs[s], out_hbm.at[addr], osems[s])

        def wait_out(s):
            pltpu.make_async_copy(
                outrs[s], out_hbm.at[NPAD + NW - 1], osems[s]).wait()

        def adv(d, j):
            sv = startv[pl.ds(d, 16)]
            s0 = sv[0]
            s1 = sv[1]
            s2 = sv[2]
            deg = s1 - s0
            nb = jnp.maximum((deg + 15) // 16, 1)
            w = ((j + 1) >= nb).astype(jnp.int32)
            dn = d + w
            jn = (1 - w) * (j + 1)
            e0n = w * s1 + (1 - w) * (s0 + 16 * (j + 1))
            degn = w * (s2 - s1) + (1 - w) * deg
            cntn = jnp.clip(degn - 16 * jn, 0, 16)
            return dn, jn, e0n, cntn

        def cbody(i, acc):
            a = startv[pl.ds(i * 16, 16)]
            b2 = startv[pl.ds(i * 16 + 1, 16)]
            deg = b2 - a
            return acc + jnp.maximum((deg + 15) // 16, 1)

        nbv = lax.fori_loop(0, DPW // 16, cbody,
                            jnp.zeros((16,), jnp.int32))
        nbtot = _bsum(nbv)[0]
        npair = (nbtot + 1) // 2

        sv0 = startv[pl.ds(0, 16)]
        c0_e0 = sv0[0]
        c0_cnt = jnp.clip(sv0[1] - sv0[0], 0, 16)
        zero_i = c0_e0 - c0_e0
        c = (zero_i, zero_i, c0_e0, c0_cnt)
        n1 = adv(c[0], c[1])
        n2 = adv(n1[0], n1[1])
        issue_idx(c[2], 0)
        issue_idx(n1[2], 1)
        if dbuf:
            wait_idx(0)
            off = c[2] - (c[2] // 8) * 8
            idx = idxb0[pl.ds(off, 16)]
            idx = jnp.where(lane < c[3], idx, 0)
            issue_rows(idx, 0)
        issue_xr(d0 + jnp.minimum(c[0], DPW - 1), 0)
        issue_out(trash, 0)
        issue_out(trash, 1)

        neg = jnp.full((16,), -1e30, jnp.float32)
        zero = jnp.zeros((16,), jnp.float32)

        def zacc(i, cc):
            accv[pl.ds(i * 16, 16)] = zero
            return cc

        lax.fori_loop(0, hcp // 16, zacc, 0)

        def block(b, s, st):
            c, n1, n2, ms, ss = st
            t = 1 - s
            c_d, c_j, c_e0, c_cnt = c
            n1_d, n1_j, n1_e0, n1_cnt = n1

            if not dbuf:
                wait_idx(s)
                off = c_e0 - (c_e0 // 8) * 8
                idxc = idxbs[s][pl.ds(off, 16)]
                idxc = jnp.where(lane < c_cnt, idxc, 0)

            issue_idx(n2[2], s)

            if dbuf:
                wait_idx(t)
                off1 = n1_e0 - (n1_e0 // 8) * 8
                idx1 = idxbs[t][pl.ds(off1, 16)]
                idx1 = jnp.where(lane < n1_cnt, idx1, 0)
                issue_rows(idx1, t)
            else:
                issue_rows(idxc, 0)

            issue_xr(d0 + jnp.minimum(n1_d, DPW - 1), t)

            wait_rows(s if dbuf else 0)
            wait_xr(s)
            xlg_s = xlgs[s]
            xrv_s = xrvs[s]
            firstv = jnp.full((16,), c_j, jnp.int32) == 0
            live = lane < c_cnt
            new_ms = []
            new_ss = []
            for h in range(HEADS):
                hb = h * cp
                ms_h = jnp.where(firstv, neg, ms[h])
                ss_h = jnp.where(firstv, zero, ss[h])

                def lbody(cb, parts, hb=hb, h=h):
                    base = hb + cb * 16
                    xr_b = xrv_s[pl.ds(base, 16)]
                    at_b = attv[h, pl.ds(cb * 16, 16)]
                    ap = at_b * a_pos
                    an = at_b * a_neg
                    out = []
                    for j in range(16):
                        tt = xlg_s[j, pl.ds(base, 16)] + xr_b
                        out.append(parts[j] + ap * tt + an * jnp.abs(tt))
                    return tuple(out)

                parts = lax.fori_loop(
                    0, nbc, lbody,
                    tuple(jnp.zeros((16,), jnp.float32) for _ in range(16)))
                for j in range(16):
                    partsv[pl.ds(j * 16, 16)] = parts[j]
                logits = jnp.zeros((16,), jnp.float32)
                for ccc in range(16):
                    logits = logits + plsc.load_gather(
                        partsv, [lane * 16 + ccc])
                logits = jnp.where(live, logits, neg)
                m_new = jnp.maximum(ms_h, _bmax(logits))
                scale = jnp.exp(ms_h - m_new)
                w = jnp.exp(logits - m_new)
                w = jnp.where(live, w, zero)
                s_new = ss_h * scale + _bsum(w)
                wj = [
                    _shuf(w, jnp.full((16,), j, jnp.int32))
                    for j in range(16)
                ]

                def abody(cb, cc, hb=hb, scale=scale, wj=wj):
                    base = hb + cb * 16
                    a = accv[pl.ds(base, 16)] * scale
                    for j in range(16):
                        a = a + wj[j] * xlg_s[j, pl.ds(base, 16)]
                    accv[pl.ds(base, 16)] = a
                    return cc

                lax.fori_loop(0, nbc, abody, 0)
                new_ms.append(m_new)
                new_ss.append(s_new)

            inv = [1.0 / (new_ss[h] + 1e-16) for h in range(HEADS)]

            wait_out(s)
            outr_s = outrs[s]

            def obody(cb, cc):
                base = cb * 16
                r = biasv[pl.ds(base, 16)]
                for h in range(HEADS):
                    r = r + 0.25 * inv[h] * accv[pl.ds(h * cp + base, 16)]
                if do_relu:
                    r = jnp.maximum(r, 0.0)
                outr_s[pl.ds(base, 16)] = r
                return cc

            lax.fori_loop(0, nbc, obody, 0)
            lastv = ((n1_d != c_d) & (b < nbtot)).astype(jnp.int32)
            addr = lastv * (d0 + c_d) + (1 - lastv) * trash
            issue_out(addr, s)

            n3 = adv(n2[0], n2[1])
            return (n1, n2, n3, tuple(new_ms), tuple(new_ss))

        init_ms = tuple(neg for _ in range(HEADS))
        init_ss = tuple(zero for _ in range(HEADS))

        def pair(g, st):
            st = block(2 * g, 0, st)
            st = block(2 * g + 1, 1, st)
            return st

        lax.fori_loop(0, npair, pair, (c, n1, n2, init_ms, init_ss))

        if dbuf:
            wait_idx(1)
            wait_rows(0)
        else:
            wait_idx(0)
            wait_idx(1)
        wait_xr(0)
        wait_out(0)
        wait_out(1)

    return ek


def _prep_params(params):
    out = []
    for li, p in enumerate(params):
        fi, hfo = p["Wl"].shape
        fo = hfo // HEADS
        cp = max(fo, 32)
        fi_pad = 8 if li == 0 else fi

        def padw(w, fi=fi, fo=fo, cp=cp, fi_pad=fi_pad):
            w = w.reshape(fi, HEADS, fo)
            w = jnp.pad(w, ((0, fi_pad - fi), (0, 0), (0, cp - fo)))
            return w.reshape(fi_pad, HEADS * cp)

        def padb(b, fo=fo, cp=cp):
            b = b.reshape(HEADS, fo)
            b = jnp.pad(b, ((0, 0), (0, cp - fo)))
            return b.reshape(1, HEADS * cp)

        out.append({
            "cp": cp,
            "Wl": padw(p["Wl"]),
            "bl": padb(p["bl"]),
            "Wr": padw(p["Wr"]),
            "br": padb(p["br"]),
            "att": jnp.pad(p["att"], ((0, 0), (0, cp - fo))),
            "b": jnp.pad(p["b"], (0, cp - fo)),
        })
    return out


def kernel(x, edge_index, params):
    loops = jnp.arange(N_NODES, dtype=edge_index.dtype)
    src = jnp.concatenate([edge_index[0], loops])
    dst = jnp.concatenate([edge_index[1], loops])
    perm = jnp.argsort(dst)
    src_s = src[perm].astype(jnp.int32)
    dst_s = dst[perm].astype(jnp.int32)
    start = jnp.searchsorted(
        dst_s, jnp.arange(START_LEN, dtype=jnp.int32)).astype(jnp.int32)
    src_pad = jnp.pad(src_s, (0, SRC_LEN - E_TOT))

    h = jnp.pad(x.astype(jnp.float32),
                ((0, NPAD - N_NODES), (0, 8 - x.shape[1])))
    pp = _prep_params(params)
    for li, p in enumerate(pp):
        xl, xr = _mm_pair(h, p["Wl"], p["bl"], p["Wr"], p["br"])
        ek = _make_edge_kernel(p["cp"], li < len(pp) - 1)
        h = ek(xl, xr, src_pad, start, p["att"], p["b"])[:NPAD]
    out = _log_softmax2(h)
    return out[:N_NODES, :2]


# speedup vs baseline: 1.0236x; 1.0236x over previous
import functools

import jax
import jax.numpy as jnp
from jax import lax
from jax.experimental import pallas as pl
from jax.experimental.pallas import tpu as pltpu
from jax.experimental.pallas import tpu_sc as plsc

N_NODES = 10000
N_EDGES = 160000
E_TOT = N_EDGES + N_NODES
HEADS = 4
NEG_SLOPE = 0.2
NPAD = 10240
NW = 32
DPW = NPAD // NW
START_LEN = NW * DPW + 32
SRC_LEN = ((E_TOT + 31) // 32) * 32 + 32


def _pick_cb(n):
    for c in range(min(2048, n), 0, -128):
        if n % c == 0:
            return c
    return n


def _mm_pair(h, wl, bl, wr, br):
    m, k = h.shape
    n = wl.shape[1]
    rb = 1024
    cb = _pick_cb(n)

    def body(x_ref, wl_ref, bl_ref, wr_ref, br_ref, ol_ref, or_ref):
        x = x_ref[...]
        ol_ref[...] = (
            jnp.dot(x, wl_ref[...], preferred_element_type=jnp.float32)
            + bl_ref[...]
        )
        or_ref[...] = (
            jnp.dot(x, wr_ref[...], preferred_element_type=jnp.float32)
            + br_ref[...]
        )

    out = pl.pallas_call(
        body,
        grid=(m // rb, n // cb),
        in_specs=[
            pl.BlockSpec((rb, k), lambda i, j: (i, 0)),
            pl.BlockSpec((k, cb), lambda i, j: (0, j)),
            pl.BlockSpec((1, cb), lambda i, j: (0, j)),
            pl.BlockSpec((k, cb), lambda i, j: (0, j)),
            pl.BlockSpec((1, cb), lambda i, j: (0, j)),
        ],
        out_specs=[
            pl.BlockSpec((rb, cb), lambda i, j: (i, j)),
            pl.BlockSpec((rb, cb), lambda i, j: (i, j)),
        ],
        out_shape=[jax.ShapeDtypeStruct((m, n), jnp.float32)] * 2,
    )(h, wl, bl, wr, br)
    return out


def _log_softmax2(h):
    def body(x_ref, o_ref):
        x = x_ref[...]
        x0 = x[:, 0:1]
        x1 = x[:, 1:2]
        mx = jnp.maximum(x0, x1)
        ls = mx + jnp.log(jnp.exp(x0 - mx) + jnp.exp(x1 - mx))
        o_ref[...] = x - ls

    return pl.pallas_call(
        body,
        grid=(NPAD // 1024,),
        in_specs=[pl.BlockSpec((1024, 32), lambda i: (i, 0))],
        out_specs=pl.BlockSpec((1024, 32), lambda i: (i, 0)),
        out_shape=jax.ShapeDtypeStruct((NPAD, 32), jnp.float32),
    )(h)


@functools.lru_cache(maxsize=None)
def _make_edge_kernel(cp: int, do_relu: bool):
    hcp = HEADS * cp
    hcpe = hcp + 128
    nbc = cp // 16
    dbuf = hcp <= 2048
    info = plsc.get_sparse_core_info()
    nc = info.num_cores
    mesh = plsc.VectorSubcoreMesh(core_axis_name="c", subcore_axis_name="s")
    a_pos = 0.5 * (1.0 + NEG_SLOPE)
    xlg1_shape = (16, hcpe) if dbuf else (16, 128)

    @functools.partial(
        pl.kernel,
        out_type=jax.ShapeDtypeStruct((NPAD + NW, cp), jnp.float32),
        mesh=mesh,
        compiler_params=pltpu.CompilerParams(needs_layout_passes=False),
        scratch_types=[
            pltpu.VMEM((DPW + 32,), jnp.int32),
            pltpu.VMEM((HEADS, cp), jnp.float32),
            pltpu.VMEM((cp,), jnp.float32),
            pltpu.VMEM((32,), jnp.int32),
            pltpu.VMEM((32,), jnp.int32),
            pltpu.VMEM((16, hcpe), jnp.float32),
            pltpu.VMEM(xlg1_shape, jnp.float32),
            pltpu.VMEM((hcpe,), jnp.float32),
            pltpu.VMEM((hcpe,), jnp.float32),
            pltpu.VMEM((hcp,), jnp.float32),
            pltpu.VMEM((cp,), jnp.float32),
            pltpu.VMEM((cp,), jnp.float32),
            pltpu.VMEM((256,), jnp.float32),
            pltpu.VMEM((256,), jnp.float32),
            pltpu.SemaphoreType.DMA,
            pltpu.SemaphoreType.DMA,
            pltpu.SemaphoreType.DMA,
            pltpu.SemaphoreType.DMA,
            pltpu.SemaphoreType.DMA,
            pltpu.SemaphoreType.DMA,
            pltpu.SemaphoreType.DMA,
            pltpu.SemaphoreType.DMA,
        ],
    )
    def ek(xl_hbm, xr_hbm, src_hbm, start_hbm, att_hbm, b_hbm, out_hbm,
           startv, attv, biasv, idxb0, idxb1, xlg0, xlg1, xrv0, xrv1,
           accv, outr0, outr1, partsv, axlbuf,
           isem0, isem1, rsem0, rsem1, xsem0, xsem1, osem0, osem1):
        wid = lax.axis_index("s") * nc + lax.axis_index("c")
        d0 = wid * DPW
        trash = NPAD + wid
        pltpu.sync_copy(start_hbm.at[pl.ds(d0, DPW + 32)], startv)
        pltpu.sync_copy(att_hbm, attv)
        pltpu.sync_copy(b_hbm, biasv)
        lane = lax.iota(jnp.int32, 16)
        zidx = jnp.zeros((16,), jnp.int32)
        dnums = lax.GatherDimensionNumbers(
            offset_dims=(), collapsed_slice_dims=(0,), start_index_map=(0,))

        idxbs = (idxb0, idxb1)
        xlgs = (xlg0, xlg1) if dbuf else (xlg0, xlg0)
        xrvs = (xrv0, xrv1)
        outrs = (outr0, outr1)
        isems = (isem0, isem1)
        rsems = (rsem0, rsem1)
        xsems = (xsem0, xsem1)
        osems = (osem0, osem1)

        def _shuf(v, idx):
            return lax.gather(
                v, idx.reshape(16, 1), dnums, slice_sizes=(1,),
                mode=lax.GatherScatterMode.PROMISE_IN_BOUNDS)

        def _bsum(v):
            for kk in (8, 4, 2, 1):
                v = v + _shuf(v, lane ^ kk)
            return v

        def _bmax(v):
            for kk in (8, 4, 2, 1):
                v = jnp.maximum(v, _shuf(v, lane ^ kk))
            return v

        def issue_idx(e0, s):
            eba = pl.multiple_of((e0 // 8) * 8, 8)
            pltpu.async_copy(src_hbm.at[pl.ds(eba, 32)], idxbs[s], isems[s])

        def wait_idx(s):
            pltpu.make_async_copy(
                src_hbm.at[pl.ds(0, 32)], idxbs[s], isems[s]).wait()

        def issue_rows(idx, s):
            pltpu.async_copy(xl_hbm.at[idx], xlgs[s], rsems[s])

        def wait_rows(s):
            pltpu.make_async_copy(xl_hbm.at[zidx], xlgs[s], rsems[s]).wait()

        def issue_xr(d, s):
            pltpu.async_copy(xr_hbm.at[d], xrvs[s], xsems[s])

        def wait_xr(s):
            pltpu.make_async_copy(xr_hbm.at[0], xrvs[s], xsems[s]).wait()

        def issue_out(addr, s):
            pltpu.async_copy(outrs[s], out_hbm.at[addr], osems[s])

        def wait_out(s):
            pltpu.make_async_copy(
                outrs[s], out_hbm.at[NPAD + NW - 1], osems[s]).wait()

        def adv(d, j):
            sv = startv[pl.ds(d, 16)]
            s0 = sv[0]
            s1 = sv[1]
            s2 = sv[2]
            deg = s1 - s0
            nb = jnp.maximum((deg + 15) // 16, 1)
            w = ((j + 1) >= nb).astype(jnp.int32)
            dn = d + w
            jn = (1 - w) * (j + 1)
            e0n = w * s1 + (1 - w) * (s0 + 16 * (j + 1))
            degn = w * (s2 - s1) + (1 - w) * deg
            cntn = jnp.clip(degn - 16 * jn, 0, 16)
            return dn, jn, e0n, cntn

        def cbody(i, acc):
            a = startv[pl.ds(i * 16, 16)]
            b2 = startv[pl.ds(i * 16 + 1, 16)]
            deg = b2 - a
            return acc + jnp.maximum((deg + 15) // 16, 1)

        nbv = lax.fori_loop(0, DPW // 16, cbody,
                            jnp.zeros((16,), jnp.int32))
        nbtot = _bsum(nbv)[0]
        npair = (nbtot + 1) // 2

        sv0 = startv[pl.ds(0, 16)]
        c0_e0 = sv0[0]
        c0_cnt = jnp.clip(sv0[1] - sv0[0], 0, 16)
        zero_i = c0_e0 - c0_e0
        c = (zero_i, zero_i, c0_e0, c0_cnt)
        n1 = adv(c[0], c[1])
        n2 = adv(n1[0], n1[1])
        issue_idx(c[2], 0)
        issue_idx(n1[2], 1)
        if dbuf:
            wait_idx(0)
            off = c[2] - (c[2] // 8) * 8
            idx = idxb0[pl.ds(off, 16)]
            idx = jnp.where(lane < c[3], idx, 0)
            issue_rows(idx, 0)
        issue_xr(d0 + jnp.minimum(c[0], DPW - 1), 0)
        issue_out(trash, 0)
        issue_out(trash, 1)

        neg = jnp.full((16,), -1e30, jnp.float32)
        zero = jnp.zeros((16,), jnp.float32)

        def zacc(i, cc):
            accv[pl.ds(i * 16, 16)] = zero
            return cc

        lax.fori_loop(0, hcp // 16, zacc, 0)

        def block(b, s, st):
            c, n1, n2, ms, ss = st
            t = 1 - s
            c_d, c_j, c_e0, c_cnt = c
            n1_d, n1_j, n1_e0, n1_cnt = n1

            if not dbuf:
                wait_idx(s)
                off = c_e0 - (c_e0 // 8) * 8
                idxc = idxbs[s][pl.ds(off, 16)]
                idxc = jnp.where(lane < c_cnt, idxc, 0)

            issue_idx(n2[2], s)

            if dbuf:
                wait_idx(t)
                off1 = n1_e0 - (n1_e0 // 8) * 8
                idx1 = idxbs[t][pl.ds(off1, 16)]
                idx1 = jnp.where(lane < n1_cnt, idx1, 0)
                issue_rows(idx1, t)
            else:
                issue_rows(idxc, 0)

            issue_xr(d0 + jnp.minimum(n1_d, DPW - 1), t)

            wait_rows(s if dbuf else 0)
            wait_xr(s)
            xlg_s = xlgs[s]
            xrv_s = xrvs[s]
            firstv = jnp.full((16,), c_j, jnp.int32) == 0
            live = lane < c_cnt
            for j in range(16):
                axlbuf[pl.ds(j * 16, 16)] = xlg_s[j, pl.ds(hcp, 16)]
            axr_t = xrv_s[pl.ds(hcp, 16)]
            new_ms = []
            new_ss = []
            for h in range(HEADS):
                hb = h * cp
                ms_h = jnp.where(firstv, neg, ms[h])
                ss_h = jnp.where(firstv, zero, ss[h])

                def lbody(cb, parts, hb=hb, h=h):
                    base = hb + cb * 16
                    xr_b = xrv_s[pl.ds(base, 16)]
                    an = attv[h, pl.ds(cb * 16, 16)]
                    out = []
                    for j in range(16):
                        tt = xlg_s[j, pl.ds(base, 16)] + xr_b
                        out.append(parts[j] + an * jnp.abs(tt))
                    return tuple(out)

                parts = lax.fori_loop(
                    0, nbc, lbody,
                    tuple(jnp.zeros((16,), jnp.float32) for _ in range(16)))
                for j in range(16):
                    partsv[pl.ds(j * 16, 16)] = parts[j]
                logits = a_pos * (
                    plsc.load_gather(axlbuf, [lane * 16 + h]) + axr_t[h])
                for ccc in range(16):
                    logits = logits + plsc.load_gather(
                        partsv, [lane * 16 + ccc])
                logits = jnp.where(live, logits, neg)
                m_new = jnp.maximum(ms_h, _bmax(logits))
                scale = jnp.exp(ms_h - m_new)
                w = jnp.exp(logits - m_new)
                w = jnp.where(live, w, zero)
                s_new = ss_h * scale + _bsum(w)
                wj = [
                    _shuf(w, jnp.full((16,), j, jnp.int32))
                    for j in range(16)
                ]

                def abody(cb, cc, hb=hb, scale=scale, wj=wj):
                    base = hb + cb * 16
                    a = accv[pl.ds(base, 16)] * scale
                    for j in range(16):
                        a = a + wj[j] * xlg_s[j, pl.ds(base, 16)]
                    accv[pl.ds(base, 16)] = a
                    return cc

                lax.fori_loop(0, nbc, abody, 0)
                new_ms.append(m_new)
                new_ss.append(s_new)

            inv = [1.0 / (new_ss[h] + 1e-16) for h in range(HEADS)]

            wait_out(s)
            outr_s = outrs[s]

            def obody(cb, cc):
                base = cb * 16
                r = biasv[pl.ds(base, 16)]
                for h in range(HEADS):
                    r = r + 0.25 * inv[h] * accv[pl.ds(h * cp + base, 16)]
                if do_relu:
                    r = jnp.maximum(r, 0.0)
                outr_s[pl.ds(base, 16)] = r
                return cc

            lax.fori_loop(0, nbc, obody, 0)
            lastv = ((n1_d != c_d) & (b < nbtot)).astype(jnp.int32)
            addr = lastv * (d0 + c_d) + (1 - lastv) * trash
            issue_out(addr, s)

            n3 = adv(n2[0], n2[1])
            return (n1, n2, n3, tuple(new_ms), tuple(new_ss))

        init_ms = tuple(neg for _ in range(HEADS))
        init_ss = tuple(zero for _ in range(HEADS))

        def pair(g, st):
            st = block(2 * g, 0, st)
            st = block(2 * g + 1, 1, st)
            return st

        lax.fori_loop(0, npair, pair, (c, n1, n2, init_ms, init_ss))

        if dbuf:
            wait_idx(1)
            wait_rows(0)
        else:
            wait_idx(0)
            wait_idx(1)
        wait_xr(0)
        wait_out(0)
        wait_out(1)

    return ek


def _prep_params(params):
    a_neg = 0.5 * (1.0 - NEG_SLOPE)
    out = []
    for li, p in enumerate(params):
        fi, hfo = p["Wl"].shape
        fo = hfo // HEADS
        cp = max(fo, 32)
        fi_pad = 8 if li == 0 else fi

        def padw(w, fi=fi, fo=fo, cp=cp, fi_pad=fi_pad):
            w = w.reshape(fi, HEADS, fo)
            w = jnp.pad(w, ((0, fi_pad - fi), (0, 0), (0, cp - fo)))
            return w.reshape(fi_pad, HEADS * cp)

        def padb(b, fo=fo, cp=cp):
            b = b.reshape(HEADS, fo)
            b = jnp.pad(b, ((0, 0), (0, cp - fo)))
            return b.reshape(1, HEADS * cp)

        attp = jnp.pad(p["att"], ((0, 0), (0, cp - fo)))
        amat = jnp.zeros((HEADS * cp, 128), jnp.float32)
        for h in range(HEADS):
            amat = amat.at[h * cp:(h + 1) * cp, h].set(attp[h])
        wl = padw(p["Wl"])
        bl = padb(p["bl"])
        wr = padw(p["Wr"])
        br = padb(p["br"])
        out.append({
            "cp": cp,
            "Wl": jnp.concatenate([wl, wl @ amat], axis=1),
            "bl": jnp.concatenate([bl, bl @ amat], axis=1),
            "Wr": jnp.concatenate([wr, wr @ amat], axis=1),
            "br": jnp.concatenate([br, br @ amat], axis=1),
            "att": a_neg * attp,
            "b": jnp.pad(p["b"], (0, cp - fo)),
        })
    return out


def kernel(x, edge_index, params):
    loops = jnp.arange(N_NODES, dtype=edge_index.dtype)
    src = jnp.concatenate([edge_index[0], loops])
    dst = jnp.concatenate([edge_index[1], loops])
    perm = jnp.argsort(dst)
    src_s = src[perm].astype(jnp.int32)
    dst_s = dst[perm].astype(jnp.int32)
    start = jnp.searchsorted(
        dst_s, jnp.arange(START_LEN, dtype=jnp.int32)).astype(jnp.int32)
    src_pad = jnp.pad(src_s, (0, SRC_LEN - E_TOT))

    h = jnp.pad(x.astype(jnp.float32),
                ((0, NPAD - N_NODES), (0, 8 - x.shape[1])))
    pp = _prep_params(params)
    for li, p in enumerate(pp):
        xl, xr = _mm_pair(h, p["Wl"], p["bl"], p["Wr"], p["br"])
        ek = _make_edge_kernel(p["cp"], li < len(pp) - 1)
        h = ek(xl, xr, src_pad, start, p["att"], p["b"])[:NPAD]
    out = _log_softmax2(h)
    return out[:N_NODES, :2]
